# baseline (speedup 1.0000x reference)
import functools

import jax
import jax.numpy as jnp
from jax import lax
from jax.experimental import pallas as pl
from jax.experimental.pallas import tpu as pltpu
from jax.experimental.pallas import tpu_sc as plsc

_VOCAB = 1000000
_EMB = 64
_RANK = 1
_WORLD = 4
_PART = _VOCAB // _WORLD
_START = _RANK * _PART
_END = _START + _PART

_R = 4096
_C = 200
_NW = 32
_RPW = _R // _NW
_CR = 4
_N = _CR * _C
_CHUNKS = _RPW // _CR
_L = 16

_GATHERS = [(o, 128) for o in range(0, _N - 128 + 1, 128)]
if _N % 128:
    _GATHERS.append((_N - _N % 128, _N % 128))

_SLICES = list(range(0, _C - _L + 1, _L))
if _C % _L:
    _SLICES.append(_C - _L)


def _body(ids_hbm, w_hbm, out_hbm, idr_v, cidx_v, scale_v, rows_v, sem):
    wid = lax.axis_index("s") * 2 + lax.axis_index("c")
    base_row = wid * _RPW

    def chunk_body(chunk, _):
        row0 = base_row + chunk * _CR
        pltpu.sync_copy(ids_hbm.at[pl.ds(row0, _CR)], idr_v)

        for rr in range(_CR):
            for o in _SLICES:
                v = idr_v[rr, pl.ds(o, _L)]
                valid = (v >= _START) & (v < _END)
                cidx_v[pl.ds(rr * _C + o, _L)] = jnp.where(
                    valid, v - _START, lax.shift_right_logical(v, 2))
                scale_v[pl.ds(rr * _C + o, _L)] = jnp.where(
                    valid, jnp.float32(1.0), jnp.float32(0.0))

        copies = [
            pltpu.async_copy(
                w_hbm.at[cidx_v.at[pl.ds(o, n)]], rows_v.at[pl.ds(o, n)], sem)
            for o, n in _GATHERS
        ]
        for c in copies:
            c.wait()

        def mul_body(g, _):
            rbase = g * _L
            s16 = scale_v[pl.ds(rbase, _L)]
            for i in range(_L):
                s = s16[i]
                for q in range(_EMB // _L):
                    sl = rows_v[rbase + i, pl.ds(q * _L, _L)]
                    rows_v[rbase + i, pl.ds(q * _L, _L)] = sl * s
            return 0

        lax.fori_loop(0, _N // _L, mul_body, 0)

        for rr in range(_CR):
            pltpu.sync_copy(
                rows_v.at[pl.ds(rr * _C, _C)], out_hbm.at[row0 + rr])
        return 0

    lax.fori_loop(0, _CHUNKS, chunk_body, 0)


@jax.jit
def _sc_lookup(ids, weight):
    kern = functools.partial(
        pl.kernel,
        out_type=jax.ShapeDtypeStruct((_R, _C, _EMB), jnp.float32),
        mesh=plsc.VectorSubcoreMesh(core_axis_name="c", subcore_axis_name="s"),
        scratch_types=[
            pltpu.VMEM((_CR, _C), jnp.int32),
            pltpu.VMEM((_N,), jnp.int32),
            pltpu.VMEM((_N,), jnp.float32),
            pltpu.VMEM((_N, _EMB), jnp.float32),
            pltpu.SemaphoreType.DMA,
        ],
        compiler_params=pltpu.CompilerParams(use_tc_tiling_on_sc=False),
    )(_body)
    return kern(ids, weight)


def kernel(input_ids, weight):
    return _sc_lookup(input_ids, weight)


# speedup vs baseline: 1.4358x; 1.4358x over previous
import functools

import jax
import jax.numpy as jnp
from jax import lax
from jax.experimental import pallas as pl
from jax.experimental.pallas import tpu as pltpu
from jax.experimental.pallas import tpu_sc as plsc

_VOCAB = 1000000
_EMB = 64
_RANK = 1
_WORLD = 4
_PART = _VOCAB // _WORLD
_START = _RANK * _PART
_END = _START + _PART

_R = 4096
_C = 200
_CP = 256
_B = _R * _C
_NW = 32
_RPW = _R // _NW
_CR = 2
_CHUNKS = _RPW // _CR
_PAIRS = _CHUNKS // 2
_L = 16
_O2 = _C - 128

_NGA = _C // _L
_TAIL = _C - _L


def _tc_prep(ids_ref, idx_ref, scale_ref):
    x = ids_ref[...]
    valid = (x >= _START) & (x < _END)
    idx_ref[...] = jnp.where(valid, x - _START, lax.shift_right_logical(x, 2))
    scale_ref[...] = jnp.where(valid, jnp.float32(1.0), jnp.float32(0.0))


def _sc_body(idx_hbm, scale_hbm, w_hbm, out_hbm,
             i_a, i_b, s_a, s_b, r_a, r_b, gs_a, gs_b, ss_a, ss_b):
    wid = lax.axis_index("s") * 2 + lax.axis_index("c")
    base = wid * _RPW

    def load_chunk(c, i_v, s_v):
        pltpu.sync_copy(idx_hbm.at[pl.ds(base + c * _CR, _CR)], i_v)
        pltpu.sync_copy(scale_hbm.at[pl.ds(base + c * _CR, _CR)], s_v)

    def fire_gathers(i_v, r_v, sem):
        for rr in range(_CR):
            for o in (0, _O2):
                pltpu.async_copy(
                    w_hbm.at[i_v.at[rr, pl.ds(o, 128)]],
                    r_v.at[rr, pl.ds(o, 128)], sem)

    def wait_gathers(i_v, r_v, sem):
        for rr in range(_CR):
            for o in (0, _O2):
                pltpu.make_async_copy(
                    w_hbm.at[i_v.at[rr, pl.ds(o, 128)]],
                    r_v.at[rr, pl.ds(o, 128)], sem).wait()

    def fire_stores(c, r_v, sem):
        for rr in range(_CR):
            pltpu.async_copy(
                r_v.at[rr],
                out_hbm.at[pl.ds((base + c * _CR + rr) * _C, _C)], sem)

    def wait_stores(r_v, sem):
        for rr in range(_CR):
            pltpu.make_async_copy(
                r_v.at[rr], out_hbm.at[pl.ds(0, _C)], sem).wait()

    def scale_rows(s_v, r_v):
        def group(o):
            for rr in range(_CR):
                s16 = s_v[rr, pl.ds(o, _L)]
                for i in range(_L):
                    s = s16[i]
                    for q in range(_EMB // _L):
                        sl = r_v[rr, o + i, pl.ds(q * _L, _L)]
                        r_v[rr, o + i, pl.ds(q * _L, _L)] = sl * s

        def g_body(g, _):
            group(g * _L)
            return 0

        lax.fori_loop(0, _NGA, g_body, 0)
        group(_TAIL)

    load_chunk(0, i_a, s_a)
    fire_gathers(i_a, r_a, gs_a)

    def pair_body(k, _):
        c0 = 2 * k
        load_chunk(c0 + 1, i_b, s_b)

        @pl.when(k > 0)
        def _():
            wait_stores(r_b, ss_b)

        fire_gathers(i_b, r_b, gs_b)

        wait_gathers(i_a, r_a, gs_a)
        scale_rows(s_a, r_a)
        fire_stores(c0, r_a, ss_a)

        wait_gathers(i_b, r_b, gs_b)
        scale_rows(s_b, r_b)
        fire_stores(c0 + 1, r_b, ss_b)

        @pl.when(k < _PAIRS - 1)
        def _():
            load_chunk(c0 + 2, i_a, s_a)
            wait_stores(r_a, ss_a)
            fire_gathers(i_a, r_a, gs_a)

        return 0

    lax.fori_loop(0, _PAIRS, pair_body, 0)
    wait_stores(r_a, ss_a)
    wait_stores(r_b, ss_b)


@jax.jit
def _sc_lookup(input_ids, weight):
    idx2, scale2 = pl.pallas_call(
        _tc_prep,
        grid=(32,),
        in_specs=[pl.BlockSpec((128, _CP), lambda i: (i, 0))],
        out_specs=[pl.BlockSpec((128, _CP), lambda i: (i, 0)),
                   pl.BlockSpec((128, _CP), lambda i: (i, 0))],
        out_shape=[jax.ShapeDtypeStruct((_R, _CP), jnp.int32),
                   jax.ShapeDtypeStruct((_R, _CP), jnp.float32)],
    )(input_ids)

    kern = functools.partial(
        pl.kernel,
        out_type=jax.ShapeDtypeStruct((_B, _EMB), jnp.float32),
        mesh=plsc.VectorSubcoreMesh(core_axis_name="c", subcore_axis_name="s"),
        scratch_types=[
            pltpu.VMEM((_CR, _CP), jnp.int32),
            pltpu.VMEM((_CR, _CP), jnp.int32),
            pltpu.VMEM((_CR, _CP), jnp.float32),
            pltpu.VMEM((_CR, _CP), jnp.float32),
            pltpu.VMEM((_CR, _C, _EMB), jnp.float32),
            pltpu.VMEM((_CR, _C, _EMB), jnp.float32),
            pltpu.SemaphoreType.DMA,
            pltpu.SemaphoreType.DMA,
            pltpu.SemaphoreType.DMA,
            pltpu.SemaphoreType.DMA,
        ],
        compiler_params=pltpu.CompilerParams(use_tc_tiling_on_sc=False),
    )(_sc_body)
    return kern(idx2, scale2, weight)


def kernel(input_ids, weight):
    out = _sc_lookup(input_ids, weight)
    return out.reshape(_R, _C, _EMB)


# speedup vs baseline: 1.4697x; 1.0236x over previous
import functools

import jax
import jax.numpy as jnp
from jax import lax
from jax.experimental import pallas as pl
from jax.experimental.pallas import tpu as pltpu
from jax.experimental.pallas import tpu_sc as plsc

_VOCAB = 1000000
_EMB = 64
_RANK = 1
_WORLD = 4
_PART = _VOCAB // _WORLD
_START = _RANK * _PART
_END = _START + _PART

_R = 4096
_C = 200
_B = _R * _C
_NW = 32
_RPW = _R // _NW
_CR = 2
_CHUNKS = _RPW // _CR
_PAIRS = _CHUNKS // 2
_L = 16
_O2 = _C - 128

_SLICES = list(range(0, _C - _L + 1, _L)) + ([_C - _L] if _C % _L else [])
_NGA = _C // _L
_TAIL = _C - _L


def _sc_body(ids_hbm, w_hbm, out_hbm,
             d_a, d_b, i_a, i_b, s_a, s_b, r_a, r_b,
             gs_a, gs_b, ss_a, ss_b):
    wid = lax.axis_index("s") * 2 + lax.axis_index("c")
    base = wid * _RPW

    def load_clamp(c, d_v, i_v, s_v):
        pltpu.sync_copy(ids_hbm.at[pl.ds(base + c * _CR, _CR)], d_v)
        for rr in range(_CR):
            for o in _SLICES:
                v = d_v[rr, pl.ds(o, _L)]
                valid = (v >= _START) & (v < _END)
                i_v[rr, pl.ds(o, _L)] = jnp.where(
                    valid, v - _START, lax.shift_right_logical(v, 2))
                s_v[rr, pl.ds(o, _L)] = jnp.where(
                    valid, jnp.float32(1.0), jnp.float32(0.0))

    def fire_gathers(i_v, r_v, sem):
        for rr in range(_CR):
            for o in (0, _O2):
                pltpu.async_copy(
                    w_hbm.at[i_v.at[rr, pl.ds(o, 128)]],
                    r_v.at[rr, pl.ds(o, 128)], sem)

    def wait_gathers(i_v, r_v, sem):
        for rr in range(_CR):
            for o in (0, _O2):
                pltpu.make_async_copy(
                    w_hbm.at[i_v.at[rr, pl.ds(o, 128)]],
                    r_v.at[rr, pl.ds(o, 128)], sem).wait()

    def fire_stores(c, r_v, sem):
        for rr in range(_CR):
            pltpu.async_copy(
                r_v.at[rr],
                out_hbm.at[pl.ds((base + c * _CR + rr) * _C, _C)], sem)

    def wait_stores(r_v, sem):
        for rr in range(_CR):
            pltpu.make_async_copy(
                r_v.at[rr], out_hbm.at[pl.ds(0, _C)], sem).wait()

    def scale_rows(s_v, r_v):
        def group(o):
            for rr in range(_CR):
                s16 = s_v[rr, pl.ds(o, _L)]
                for i in range(_L):
                    s = s16[i]
                    for q in range(_EMB // _L):
                        sl = r_v[rr, o + i, pl.ds(q * _L, _L)]
                        r_v[rr, o + i, pl.ds(q * _L, _L)] = sl * s

        def g_body(g, _):
            group(g * _L)
            return 0

        lax.fori_loop(0, _NGA, g_body, 0)
        group(_TAIL)

    load_clamp(0, d_a, i_a, s_a)
    fire_gathers(i_a, r_a, gs_a)

    def pair_body(k, _):
        c0 = 2 * k
        load_clamp(c0 + 1, d_b, i_b, s_b)

        @pl.when(k > 0)
        def _():
            wait_stores(r_b, ss_b)

        fire_gathers(i_b, r_b, gs_b)

        wait_gathers(i_a, r_a, gs_a)
        scale_rows(s_a, r_a)
        fire_stores(c0, r_a, ss_a)

        wait_gathers(i_b, r_b, gs_b)
        scale_rows(s_b, r_b)
        fire_stores(c0 + 1, r_b, ss_b)

        @pl.when(k < _PAIRS - 1)
        def _():
            load_clamp(c0 + 2, d_a, i_a, s_a)
            wait_stores(r_a, ss_a)
            fire_gathers(i_a, r_a, gs_a)

        return 0

    lax.fori_loop(0, _PAIRS, pair_body, 0)
    wait_stores(r_a, ss_a)
    wait_stores(r_b, ss_b)


@jax.jit
def _sc_lookup(input_ids, weight):
    kern = functools.partial(
        pl.kernel,
        out_type=jax.ShapeDtypeStruct((_B, _EMB), jnp.float32),
        mesh=plsc.VectorSubcoreMesh(core_axis_name="c", subcore_axis_name="s"),
        scratch_types=[
            pltpu.VMEM((_CR, _C), jnp.int32),
            pltpu.VMEM((_CR, _C), jnp.int32),
            pltpu.VMEM((_CR, _C), jnp.int32),
            pltpu.VMEM((_CR, _C), jnp.int32),
            pltpu.VMEM((_CR, _C), jnp.float32),
            pltpu.VMEM((_CR, _C), jnp.float32),
            pltpu.VMEM((_CR, _C, _EMB), jnp.float32),
            pltpu.VMEM((_CR, _C, _EMB), jnp.float32),
            pltpu.SemaphoreType.DMA,
            pltpu.SemaphoreType.DMA,
            pltpu.SemaphoreType.DMA,
            pltpu.SemaphoreType.DMA,
        ],
        compiler_params=pltpu.CompilerParams(use_tc_tiling_on_sc=False),
    )(_sc_body)
    return kern(input_ids, weight)


def kernel(input_ids, weight):
    out = _sc_lookup(input_ids, weight)
    return out.reshape(_R, _C, _EMB)


# speedup vs baseline: 1.4757x; 1.0041x over previous
import functools

import jax
import jax.numpy as jnp
from jax import lax
from jax.experimental import pallas as pl
from jax.experimental.pallas import tpu as pltpu
from jax.experimental.pallas import tpu_sc as plsc

_VOCAB = 1000000
_EMB = 64
_RANK = 1
_WORLD = 4
_PART = _VOCAB // _WORLD
_START = _RANK * _PART
_END = _START + _PART

_R = 4096
_C = 200
_B = _R * _C
_NW = 32
_RPW = _R // _NW
_CR = 4
_CHUNKS = _RPW // _CR
_PAIRS = _CHUNKS // 2
_L = 16
_O2 = _C - 128

_SLICES = list(range(0, _C - _L + 1, _L)) + ([_C - _L] if _C % _L else [])
_NGA = _C // _L
_TAIL = _C - _L


def _sc_body(ids_hbm, w_hbm, out_hbm,
             d_a, d_b, i_a, i_b, s_a, s_b, r_a, r_b,
             gs_a, gs_b, ss_a, ss_b):
    wid = lax.axis_index("s") * 2 + lax.axis_index("c")
    base = wid * _RPW

    def load_clamp(c, d_v, i_v, s_v):
        pltpu.sync_copy(ids_hbm.at[pl.ds(base + c * _CR, _CR)], d_v)
        for rr in range(_CR):
            for o in _SLICES:
                v = d_v[rr, pl.ds(o, _L)]
                valid = (v >= _START) & (v < _END)
                i_v[rr, pl.ds(o, _L)] = jnp.where(
                    valid, v - _START, lax.shift_right_logical(v, 2))
                s_v[rr, pl.ds(o, _L)] = jnp.where(
                    valid, jnp.float32(1.0), jnp.float32(0.0))

    def fire_gathers(i_v, r_v, sem):
        for rr in range(_CR):
            for o in (0, _O2):
                pltpu.async_copy(
                    w_hbm.at[i_v.at[rr, pl.ds(o, 128)]],
                    r_v.at[rr, pl.ds(o, 128)], sem)

    def wait_gathers(i_v, r_v, sem):
        for rr in range(_CR):
            for o in (0, _O2):
                pltpu.make_async_copy(
                    w_hbm.at[i_v.at[rr, pl.ds(o, 128)]],
                    r_v.at[rr, pl.ds(o, 128)], sem).wait()

    def fire_stores(c, r_v, sem):
        for rr in range(_CR):
            pltpu.async_copy(
                r_v.at[rr],
                out_hbm.at[pl.ds((base + c * _CR + rr) * _C, _C)], sem)

    def wait_stores(r_v, sem):
        for rr in range(_CR):
            pltpu.make_async_copy(
                r_v.at[rr], out_hbm.at[pl.ds(0, _C)], sem).wait()

    def scale_rows(s_v, r_v):
        def group(o):
            for rr in range(_CR):
                s16 = s_v[rr, pl.ds(o, _L)]
                for i in range(_L):
                    s = s16[i]
                    for q in range(_EMB // _L):
                        sl = r_v[rr, o + i, pl.ds(q * _L, _L)]
                        r_v[rr, o + i, pl.ds(q * _L, _L)] = sl * s

        def g_body(g, _):
            group(g * _L)
            return 0

        lax.fori_loop(0, _NGA, g_body, 0)
        group(_TAIL)

    load_clamp(0, d_a, i_a, s_a)
    fire_gathers(i_a, r_a, gs_a)

    def pair_body(k, _):
        c0 = 2 * k
        load_clamp(c0 + 1, d_b, i_b, s_b)

        @pl.when(k > 0)
        def _():
            wait_stores(r_b, ss_b)

        fire_gathers(i_b, r_b, gs_b)

        wait_gathers(i_a, r_a, gs_a)
        scale_rows(s_a, r_a)
        fire_stores(c0, r_a, ss_a)

        wait_gathers(i_b, r_b, gs_b)
        scale_rows(s_b, r_b)
        fire_stores(c0 + 1, r_b, ss_b)

        @pl.when(k < _PAIRS - 1)
        def _():
            load_clamp(c0 + 2, d_a, i_a, s_a)
            wait_stores(r_a, ss_a)
            fire_gathers(i_a, r_a, gs_a)

        return 0

    lax.fori_loop(0, _PAIRS, pair_body, 0)
    wait_stores(r_a, ss_a)
    wait_stores(r_b, ss_b)


@jax.jit
def _sc_lookup(input_ids, weight):
    kern = functools.partial(
        pl.kernel,
        out_type=jax.ShapeDtypeStruct((_B, _EMB), jnp.float32),
        mesh=plsc.VectorSubcoreMesh(core_axis_name="c", subcore_axis_name="s"),
        scratch_types=[
            pltpu.VMEM((_CR, _C), jnp.int32),
            pltpu.VMEM((_CR, _C), jnp.int32),
            pltpu.VMEM((_CR, _C), jnp.int32),
            pltpu.VMEM((_CR, _C), jnp.int32),
            pltpu.VMEM((_CR, _C), jnp.float32),
            pltpu.VMEM((_CR, _C), jnp.float32),
            pltpu.VMEM((_CR, _C, _EMB), jnp.float32),
            pltpu.VMEM((_CR, _C, _EMB), jnp.float32),
            pltpu.SemaphoreType.DMA,
            pltpu.SemaphoreType.DMA,
            pltpu.SemaphoreType.DMA,
            pltpu.SemaphoreType.DMA,
        ],
        compiler_params=pltpu.CompilerParams(use_tc_tiling_on_sc=False),
    )(_sc_body)
    return kern(input_ids, weight)


def kernel(input_ids, weight):
    out = _sc_lookup(input_ids, weight)
    return out.reshape(_R, _C, _EMB)
